# baseline (speedup 1.0000x reference)
import functools

import jax
import jax.numpy as jnp
from jax import lax
from jax.experimental import pallas as pl
from jax.experimental.pallas import tpu as pltpu
from jax.experimental.pallas import tpu_sc as plsc

CN = 512


def kernel(xyz, points):
    B, N, C = xyz.shape
    D = points.shape[-1]
    F = C + D
    mesh = plsc.VectorSubcoreMesh(core_axis_name="c", subcore_axis_name="s")
    nc = 2

    @functools.partial(
        pl.kernel,
        mesh=mesh,
        out_type=jax.ShapeDtypeStruct((B, N * F), xyz.dtype),
        scratch_types=[
            pltpu.VMEM((CN * C + 16,), jnp.float32),
            pltpu.VMEM((CN * D,), jnp.float32),
            pltpu.VMEM((CN * F,), jnp.float32),
        ],
        compiler_params=pltpu.CompilerParams(use_tc_tiling_on_sc=False),
    )
    def _group_all(xyz_hbm, pts_hbm, out_hbm, xbuf, pbuf, obuf):
        b = lax.axis_index("s") * nc + lax.axis_index("c")

        def chunk(i, _):
            n0 = i * CN
            pltpu.sync_copy(xyz_hbm.at[b, pl.ds(n0 * C, CN * C)], xbuf.at[pl.ds(0, CN * C)])
            pltpu.sync_copy(pts_hbm.at[b, pl.ds(n0 * D, CN * D)], pbuf)

            def row(r, _):
                base = r * F
                x = xbuf[pl.ds(r * C, 16)]
                obuf[pl.ds(base, 16)] = x
                for k in range(D // 16):
                    v = pbuf[pl.ds(r * D + k * 16, 16)]
                    obuf[pl.ds(base + C + k * 16, 16)] = v
                return ()

            lax.fori_loop(0, CN, row, ())
            pltpu.sync_copy(obuf, out_hbm.at[b, pl.ds(n0 * F, CN * F)])
            return ()

        lax.fori_loop(0, N // CN, chunk, ())

    out = _group_all(xyz.reshape(B, N * C), points.reshape(B, N * D))
    new_xyz = jnp.zeros((B, 1, C), dtype=xyz.dtype)
    return new_xyz, out.reshape(B, 1, N, F)


# speedup vs baseline: 3.7261x; 3.7261x over previous
import jax
import jax.numpy as jnp
from jax.experimental import pallas as pl


def _concat_body(xyz_ref, pts_ref, out_ref):
    out_ref[0:3] = xyz_ref[0].reshape(8, 64, 128)[0:3]
    out_ref[3:] = pts_ref[0].reshape(64, 64, 128)


def kernel(xyz, points):
    B, N, C = xyz.shape
    D = points.shape[-1]
    F = C + D
    NL = N // 128
    xyz_p = jnp.pad(jnp.transpose(xyz, (0, 2, 1)), ((0, 0), (0, 8 - C), (0, 0)))
    pts_t = jnp.transpose(points, (0, 2, 1))
    out_t = pl.pallas_call(
        _concat_body,
        grid=(B,),
        in_specs=[
            pl.BlockSpec((1, 8, N), lambda b: (b, 0, 0)),
            pl.BlockSpec((1, D, N), lambda b: (b, 0, 0)),
        ],
        out_specs=pl.BlockSpec((F, NL, 128), lambda b: (b, 0, 0)),
        out_shape=jax.ShapeDtypeStruct((B * F, NL, 128), xyz.dtype),
    )(xyz_p, pts_t)
    new_xyz = jnp.zeros((B, 1, C), dtype=xyz.dtype)
    out3 = out_t.reshape(B, F, N)
    return new_xyz, jnp.transpose(out3, (0, 2, 1)).reshape(B, 1, N, F)


# speedup vs baseline: 14.9641x; 4.0160x over previous
import jax
import jax.numpy as jnp
from jax.experimental import pallas as pl


BB = 8


def _concat_body(xyz_ref, pts_ref, out_ref):
    for i in range(BB):
        out_ref[i, 0:3, 0, :] = xyz_ref[0:3, i, :]
        out_ref[i, 3:, 0, :] = pts_ref[i]


def kernel(xyz, points):
    B, N, C = xyz.shape
    D = points.shape[-1]
    F = C + D
    xyz_t = jnp.transpose(xyz, (2, 0, 1))
    pts_t = jnp.transpose(points, (0, 2, 1))
    out_t = pl.pallas_call(
        _concat_body,
        grid=(B // BB, 2),
        in_specs=[
            pl.BlockSpec((C, BB, N // 2), lambda b, n: (0, b, n)),
            pl.BlockSpec((BB, D, N // 2), lambda b, n: (b, 0, n)),
        ],
        out_specs=pl.BlockSpec((BB, F, 1, N // 2), lambda b, n: (b, 0, 0, n)),
        out_shape=jax.ShapeDtypeStruct((B, F, 1, N), xyz.dtype),
    )(xyz_t, pts_t)
    new_xyz = jnp.zeros((B, 1, C), dtype=xyz.dtype)
    return new_xyz, jnp.transpose(out_t, (0, 2, 3, 1))
